# baseline (speedup 1.0000x reference)
import functools

import jax
import jax.numpy as jnp
from jax import lax
from jax.experimental import pallas as pl
from jax.experimental.pallas import tpu as pltpu
from jax.experimental.pallas import tpu_sc as plsc

NUM_WORKERS = 32
TOTAL = 32768
PER_W = TOTAL // NUM_WORKERS
CHUNKS = 8
CHUNK = PER_W // CHUNKS
DIM = 32
ROUNDS = 4
RT = PER_W // ROUNDS
RCHUNKS = RT // CHUNK
GROUPS = RT // 16


def _vsqrt(x):
    b = lax.bitcast_convert_type(x, jnp.int32)
    y = lax.bitcast_convert_type(
        jnp.int32(0x1FBD1DF5) + lax.shift_right_logical(b, 1), jnp.float32)
    for _ in range(3):
        y = 0.5 * (y + x / y)
    return y


def _make_kernel():
    mesh = plsc.VectorSubcoreMesh(core_axis_name="c", subcore_axis_name="s")

    @functools.partial(
        pl.kernel,
        mesh=mesh,
        out_type=jax.ShapeDtypeStruct((TOTAL,), jnp.float32),
        compiler_params=pltpu.CompilerParams(
            use_tc_tiling_on_sc=True, needs_layout_passes=False),
        scratch_types=[
            pltpu.VMEM((CHUNKS, CHUNK), jnp.int32),
            pltpu.VMEM((CHUNKS, CHUNK), jnp.int32),
            pltpu.VMEM((CHUNKS, CHUNK), jnp.int32),
            pltpu.VMEM((PER_W,), jnp.int32),
            pltpu.VMEM((PER_W,), jnp.int32),
            pltpu.VMEM((PER_W,), jnp.int32),
            pltpu.VMEM((RT, 128), jnp.float32),
            pltpu.VMEM((RT, 128), jnp.float32),
            pltpu.VMEM((RT, 128), jnp.float32),
            pltpu.VMEM((PER_W,), jnp.float32),
            pltpu.SemaphoreType.DMA,
        ],
    )
    def kern(h_row_hbm, r_row_hbm, t_row_hbm, h_off_hbm, r_off_hbm, t_off_hbm,
             ent_hbm, rel_hbm, out_hbm,
             h_row, r_row, t_row, h_off, r_off, t_off,
             h_st, r_st, t_st, out_v, sem):
        wid = lax.axis_index("s") * 2 + lax.axis_index("c")
        base = wid * PER_W

        pltpu.sync_copy(h_row_hbm.at[wid], h_row)
        pltpu.sync_copy(r_row_hbm.at[wid], r_row)
        pltpu.sync_copy(t_row_hbm.at[wid], t_row)
        pltpu.sync_copy(h_off_hbm.at[pl.ds(base, PER_W)], h_off)
        pltpu.sync_copy(r_off_hbm.at[pl.ds(base, PER_W)], r_off)
        pltpu.sync_copy(t_off_hbm.at[pl.ds(base, PER_W)], t_off)

        lane = lax.iota(jnp.int32, 16)

        for rnd in range(ROUNDS):
            copies = []
            for c in range(RCHUNKS):
                j = rnd * RCHUNKS + c
                dst = pl.ds(c * CHUNK, CHUNK)
                copies.append(pltpu.async_copy(
                    ent_hbm.at[h_row.at[j]], h_st.at[dst], sem))
                copies.append(pltpu.async_copy(
                    ent_hbm.at[t_row.at[j]], t_st.at[dst], sem))
                copies.append(pltpu.async_copy(
                    rel_hbm.at[r_row.at[j]], r_st.at[dst], sem))
            for cp in copies:
                cp.wait()

            def group(g, _):
                i0 = rnd * RT + g * 16
                loc = g * 16 + lane
                ho = h_off[pl.ds(i0, 16)]
                ro = r_off[pl.ds(i0, 16)]
                to = t_off[pl.ds(i0, 16)]
                acc = jnp.zeros((16,), jnp.float32)
                for j in range(DIM):
                    h = plsc.load_gather(h_st, [loc, ho + j])
                    r = plsc.load_gather(r_st, [loc, ro + j])
                    t = plsc.load_gather(t_st, [loc, to + j])
                    d = (h + r) - t
                    acc = acc + d * d
                out_v[pl.ds(i0, 16)] = _vsqrt(acc)
                return 0

            lax.fori_loop(0, GROUPS, group, 0)

        pltpu.sync_copy(out_v, out_hbm.at[pl.ds(base, PER_W)])

    return kern


_KERNEL = _make_kernel()


def kernel(pos_h, pos_r, pos_t, neg_h, neg_r, neg_t, entity_emb, relation_emb):
    h_idx = jnp.concatenate([pos_h, neg_h]).astype(jnp.int32)
    r_idx = jnp.concatenate([pos_r, neg_r]).astype(jnp.int32)
    t_idx = jnp.concatenate([pos_t, neg_t]).astype(jnp.int32)
    cshape = (NUM_WORKERS, CHUNKS, CHUNK)
    oshape = (TOTAL,)
    return _KERNEL(
        (h_idx >> 2).reshape(cshape), (r_idx >> 2).reshape(cshape),
        (t_idx >> 2).reshape(cshape),
        ((h_idx & 3) * 32).reshape(oshape), ((r_idx & 3) * 32).reshape(oshape),
        ((t_idx & 3) * 32).reshape(oshape),
        entity_emb.reshape(250000, 128), relation_emb.reshape(250, 128))


# speedup vs baseline: 1.0709x; 1.0709x over previous
import functools

import jax
import jax.numpy as jnp
from jax import lax
from jax.experimental import pallas as pl
from jax.experimental.pallas import tpu as pltpu
from jax.experimental.pallas import tpu_sc as plsc

NUM_WORKERS = 32
TOTAL = 32768
PER_W = TOTAL // NUM_WORKERS
CHUNKS = 8
CHUNK = PER_W // CHUNKS
DIM = 32
ROUNDS = 4
RT = PER_W // ROUNDS
RCHUNKS = RT // CHUNK
GROUPS = RT // 16


def _vsqrt(x):
    b = lax.bitcast_convert_type(x, jnp.int32)
    y = lax.bitcast_convert_type(
        jnp.int32(0x1FBD1DF5) + lax.shift_right_logical(b, 1), jnp.float32)
    for _ in range(3):
        y = 0.5 * (y + x / y)
    return y


def _make_kernel():
    mesh = plsc.VectorSubcoreMesh(core_axis_name="c", subcore_axis_name="s")

    @functools.partial(
        pl.kernel,
        mesh=mesh,
        out_type=jax.ShapeDtypeStruct((TOTAL,), jnp.float32),
        compiler_params=pltpu.CompilerParams(
            use_tc_tiling_on_sc=True, needs_layout_passes=False),
        scratch_types=[
            pltpu.VMEM((CHUNKS, CHUNK), jnp.int32),
            pltpu.VMEM((CHUNKS, CHUNK), jnp.int32),
            pltpu.VMEM((CHUNKS, CHUNK), jnp.int32),
            pltpu.VMEM((RT, 128), jnp.float32),
            pltpu.VMEM((RT, 128), jnp.float32),
            pltpu.VMEM((RT, 128), jnp.float32),
            pltpu.VMEM((PER_W,), jnp.float32),
            pltpu.SemaphoreType.DMA,
        ],
    )
    def kern(h_row_hbm, r_row_hbm, t_row_hbm, ent_hbm, rel_hbm, out_hbm,
             h_row, r_row, t_row, h_st, r_st, t_st, out_v, sem):
        wid = lax.axis_index("s") * 2 + lax.axis_index("c")
        base = wid * PER_W

        pltpu.sync_copy(h_row_hbm.at[wid], h_row)
        pltpu.sync_copy(r_row_hbm.at[wid], r_row)
        pltpu.sync_copy(t_row_hbm.at[wid], t_row)

        lane = lax.iota(jnp.int32, 16)

        for rnd in range(ROUNDS):
            copies = []
            for c in range(RCHUNKS):
                j = rnd * RCHUNKS + c
                dst = pl.ds(c * CHUNK, CHUNK)
                copies.append(pltpu.async_copy(
                    ent_hbm.at[h_row.at[j]], h_st.at[dst], sem))
                copies.append(pltpu.async_copy(
                    ent_hbm.at[t_row.at[j]], t_st.at[dst], sem))
                copies.append(pltpu.async_copy(
                    rel_hbm.at[r_row.at[j]], r_st.at[dst], sem))
            for cp in copies:
                cp.wait()

            def group(g, _):
                i0 = rnd * RT + g * 16
                loc = g * 16 + lane
                acc = jnp.zeros((16,), jnp.float32)
                for j in range(DIM):
                    cj = jnp.full((16,), j, jnp.int32)
                    h = plsc.load_gather(h_st, [loc, cj])
                    r = plsc.load_gather(r_st, [loc, cj])
                    t = plsc.load_gather(t_st, [loc, cj])
                    d = (h + r) - t
                    acc = acc + d * d
                out_v[pl.ds(i0, 16)] = _vsqrt(acc)
                return 0

            lax.fori_loop(0, GROUPS, group, 0)

        pltpu.sync_copy(out_v, out_hbm.at[pl.ds(base, PER_W)])

    return kern


_KERNEL = _make_kernel()


def kernel(pos_h, pos_r, pos_t, neg_h, neg_r, neg_t, entity_emb, relation_emb):
    h_idx = jnp.concatenate([pos_h, neg_h]).astype(jnp.int32)
    r_idx = jnp.concatenate([pos_r, neg_r]).astype(jnp.int32)
    t_idx = jnp.concatenate([pos_t, neg_t]).astype(jnp.int32)
    cshape = (NUM_WORKERS, CHUNKS, CHUNK)
    ent_pad = jnp.pad(entity_emb, ((0, 0), (0, 128 - DIM)))
    rel_pad = jnp.pad(relation_emb, ((0, 0), (0, 128 - DIM)))
    return _KERNEL(h_idx.reshape(cshape), r_idx.reshape(cshape),
                   t_idx.reshape(cshape), ent_pad, rel_pad)


# speedup vs baseline: 1.0763x; 1.0051x over previous
import functools

import jax
import jax.numpy as jnp
from jax import lax
from jax.experimental import pallas as pl
from jax.experimental.pallas import tpu as pltpu
from jax.experimental.pallas import tpu_sc as plsc

NUM_WORKERS = 32
TOTAL = 32768
PER_W = TOTAL // NUM_WORKERS
CHUNKS = 8
CHUNK = PER_W // CHUNKS
DIM = 32
GROUPS = PER_W // 16


def _vsqrt(x):
    b = lax.bitcast_convert_type(x, jnp.int32)
    y = lax.bitcast_convert_type(
        jnp.int32(0x1FBD1DF5) + lax.shift_right_logical(b, 1), jnp.float32)
    for _ in range(3):
        y = 0.5 * (y + x / y)
    return y


def _make_kernel():
    mesh = plsc.VectorSubcoreMesh(core_axis_name="c", subcore_axis_name="s")

    @functools.partial(
        pl.kernel,
        mesh=mesh,
        out_type=jax.ShapeDtypeStruct((TOTAL,), jnp.float32),
        compiler_params=pltpu.CompilerParams(
            use_tc_tiling_on_sc=False, needs_layout_passes=False),
        scratch_types=[
            pltpu.VMEM((CHUNKS, CHUNK), jnp.int32),
            pltpu.VMEM((CHUNKS, CHUNK), jnp.int32),
            pltpu.VMEM((CHUNKS, CHUNK), jnp.int32),
            pltpu.VMEM((PER_W, DIM), jnp.float32),
            pltpu.VMEM((PER_W, DIM), jnp.float32),
            pltpu.VMEM((PER_W, DIM), jnp.float32),
            pltpu.VMEM((PER_W,), jnp.float32),
            pltpu.SemaphoreType.DMA,
        ],
    )
    def kern(h_idx_hbm, r_idx_hbm, t_idx_hbm, ent_hbm, rel_hbm, out_hbm,
             h_idx, r_idx, t_idx, h_rows, r_rows, t_rows, out_v, sem):
        wid = lax.axis_index("s") * 2 + lax.axis_index("c")

        pltpu.sync_copy(h_idx_hbm.at[wid], h_idx)
        pltpu.sync_copy(r_idx_hbm.at[wid], r_idx)
        pltpu.sync_copy(t_idx_hbm.at[wid], t_idx)

        copies = []
        for j in range(CHUNKS):
            dst = h_rows.at[pl.ds(j * CHUNK, CHUNK)]
            copies.append(pltpu.async_copy(ent_hbm.at[h_idx.at[j]], dst, sem))
        for j in range(CHUNKS):
            dst = t_rows.at[pl.ds(j * CHUNK, CHUNK)]
            copies.append(pltpu.async_copy(ent_hbm.at[t_idx.at[j]], dst, sem))
        for j in range(CHUNKS):
            dst = r_rows.at[pl.ds(j * CHUNK, CHUNK)]
            copies.append(pltpu.async_copy(rel_hbm.at[r_idx.at[j]], dst, sem))
        for c in copies:
            c.wait()

        lane = lax.iota(jnp.int32, 16)

        def group(g, _):
            rows16 = g * 16 + lane
            acc = jnp.zeros((16,), jnp.float32)
            for j in range(DIM):
                cj = jnp.full((16,), j, jnp.int32)
                h = plsc.load_gather(h_rows, [rows16, cj])
                r = plsc.load_gather(r_rows, [rows16, cj])
                t = plsc.load_gather(t_rows, [rows16, cj])
                d = (h + r) - t
                acc = acc + d * d
            out_v[pl.ds(g * 16, 16)] = _vsqrt(acc)
            return 0

        lax.fori_loop(0, GROUPS, group, 0)

        pltpu.sync_copy(out_v, out_hbm.at[pl.ds(wid * PER_W, PER_W)])

    return kern


_KERNEL = _make_kernel()


def kernel(pos_h, pos_r, pos_t, neg_h, neg_r, neg_t, entity_emb, relation_emb):
    h_idx = jnp.concatenate([pos_h, neg_h]).astype(jnp.int32)
    r_idx = jnp.concatenate([pos_r, neg_r]).astype(jnp.int32)
    t_idx = jnp.concatenate([pos_t, neg_t]).astype(jnp.int32)
    shape = (NUM_WORKERS, CHUNKS, CHUNK)
    return _KERNEL(h_idx.reshape(shape), r_idx.reshape(shape),
                   t_idx.reshape(shape), entity_emb, relation_emb)


# speedup vs baseline: 1.7548x; 1.6304x over previous
import functools

import jax
import jax.numpy as jnp
from jax import lax
from jax.experimental import pallas as pl
from jax.experimental.pallas import tpu as pltpu
from jax.experimental.pallas import tpu_sc as plsc

NUM_WORKERS = 32
TOTAL = 32768
PER_W = TOTAL // NUM_WORKERS
CHUNKS = 8
CHUNK = PER_W // CHUNKS
DIM = 32

NUM_ENT = 1_000_000
PACK_B = 2048
SUPER = 4 * PACK_B
NSUPER = -(-NUM_ENT // SUPER)
ENT_ROWS = NSUPER * PACK_B
REL_ROWS = 256


def _vsqrt(x):
    b = lax.bitcast_convert_type(x, jnp.int32)
    y = lax.bitcast_convert_type(
        jnp.int32(0x1FBD1DF5) + lax.shift_right_logical(b, 1), jnp.float32)
    for _ in range(3):
        y = 0.5 * (y + x / y)
    return y


def _pack_block(in_ref, out_ref):
    x = in_ref[...]
    out_ref[...] = jnp.concatenate(
        [x[:, q * PACK_B:(q + 1) * PACK_B].T for q in range(4)], axis=1)


def _pack_entities(ent_t):
    return pl.pallas_call(
        _pack_block,
        grid=(NSUPER,),
        in_specs=[pl.BlockSpec((DIM, SUPER), lambda s: (0, s))],
        out_specs=pl.BlockSpec((PACK_B, 4 * DIM), lambda s: (s, 0)),
        out_shape=jax.ShapeDtypeStruct((ENT_ROWS, 4 * DIM), jnp.float32),
    )(ent_t)


def _make_sc_kernel():
    mesh = plsc.VectorSubcoreMesh(core_axis_name="c", subcore_axis_name="s")

    @functools.partial(
        pl.kernel,
        mesh=mesh,
        out_type=jax.ShapeDtypeStruct((TOTAL,), jnp.float32),
        compiler_params=pltpu.CompilerParams(
            use_tc_tiling_on_sc=True, needs_layout_passes=False),
        scratch_types=[
            pltpu.VMEM((CHUNKS, CHUNK), jnp.int32),
            pltpu.VMEM((CHUNKS, CHUNK), jnp.int32),
            pltpu.VMEM((CHUNKS, CHUNK), jnp.int32),
            pltpu.VMEM((CHUNKS, CHUNK), jnp.int32),
            pltpu.VMEM((CHUNKS, CHUNK), jnp.int32),
            pltpu.VMEM((CHUNKS, CHUNK), jnp.int32),
            pltpu.VMEM((CHUNKS, CHUNK), jnp.int32),
            pltpu.VMEM((CHUNKS, CHUNK), jnp.int32),
            pltpu.VMEM((CHUNKS, CHUNK), jnp.int32),
            pltpu.VMEM((CHUNK, 4 * DIM), jnp.float32),
            pltpu.VMEM((CHUNK, 4 * DIM), jnp.float32),
            pltpu.VMEM((CHUNK, 4 * DIM), jnp.float32),
            pltpu.VMEM((CHUNK, 4 * DIM), jnp.float32),
            pltpu.VMEM((CHUNK, 4 * DIM), jnp.float32),
            pltpu.VMEM((CHUNK, 4 * DIM), jnp.float32),
            pltpu.VMEM((PER_W,), jnp.float32),
            pltpu.SemaphoreType.DMA,
            pltpu.SemaphoreType.DMA,
        ],
    )
    def kern(h_idx_hbm, r_idx_hbm, t_idx_hbm, entp_hbm, relp_hbm, out_hbm,
             h_idx, r_idx, t_idx, h_row, r_row, t_row, h_cb, r_cb, t_cb,
             h_a, t_a, r_a, h_b, t_b, r_b, out_v, sem_a, sem_b):
        wid = lax.axis_index("s") * 2 + lax.axis_index("c")

        pltpu.sync_copy(h_idx_hbm.at[wid], h_idx)
        pltpu.sync_copy(r_idx_hbm.at[wid], r_idx)
        pltpu.sync_copy(t_idx_hbm.at[wid], t_idx)

        for j in range(CHUNKS):
            for g in range(CHUNK // 16):
                sl = pl.ds(g * 16, 16)
                for src, row, cb in ((h_idx, h_row, h_cb),
                                     (t_idx, t_row, t_cb)):
                    e = src[j, sl]
                    row[j, sl] = ((e >> 13) << 11) + (e & 2047)
                    cb[j, sl] = ((e >> 11) & 3) << 5
                e = r_idx[j, sl]
                r_row[j, sl] = e & 255
                r_cb[j, sl] = ((e >> 8) & 3) << 5

        bufs = ((h_a, t_a, r_a), (h_b, t_b, r_b))
        sems = (sem_a, sem_b)

        def fire(j):
            hb, tb, rb = bufs[j % 2]
            sem = sems[j % 2]
            return (pltpu.async_copy(entp_hbm.at[h_row.at[j]], hb, sem),
                    pltpu.async_copy(entp_hbm.at[t_row.at[j]], tb, sem),
                    pltpu.async_copy(relp_hbm.at[r_row.at[j]], rb, sem))

        lane = lax.iota(jnp.int32, 16)

        def compute(j):
            hb, tb, rb = bufs[j % 2]

            def group(g, _):
                pos = g * 16 + lane
                jv = jnp.full((16,), j, jnp.int32)
                cbh = plsc.load_gather(h_cb, [jv, pos])
                cbt = plsc.load_gather(t_cb, [jv, pos])
                cbr = plsc.load_gather(r_cb, [jv, pos])
                acc = jnp.zeros((16,), jnp.float32)
                for d in range(DIM):
                    h = plsc.load_gather(hb, [pos, cbh + d])
                    t = plsc.load_gather(tb, [pos, cbt + d])
                    r = plsc.load_gather(rb, [pos, cbr + d])
                    dd = (h + r) - t
                    acc = acc + dd * dd
                out_v[pl.ds(j * CHUNK + g * 16, 16)] = _vsqrt(acc)
                return 0

            lax.fori_loop(0, CHUNK // 16, group, 0)

        pending = fire(0)
        for j in range(1, CHUNKS):
            nxt = fire(j)
            for c in pending:
                c.wait()
            compute(j - 1)
            pending = nxt
        for c in pending:
            c.wait()
        compute(CHUNKS - 1)

        pltpu.sync_copy(out_v, out_hbm.at[pl.ds(wid * PER_W, PER_W)])

    return kern


_SC_KERNEL = _make_sc_kernel()


def kernel(pos_h, pos_r, pos_t, neg_h, neg_r, neg_t, entity_emb, relation_emb):
    h_idx = jnp.concatenate([pos_h, neg_h]).astype(jnp.int32)
    r_idx = jnp.concatenate([pos_r, neg_r]).astype(jnp.int32)
    t_idx = jnp.concatenate([pos_t, neg_t]).astype(jnp.int32)

    ent_packed = _pack_entities(entity_emb.T)

    nrel = relation_emb.shape[0]
    rel_packed = jnp.pad(relation_emb, ((0, 4 * REL_ROWS - nrel), (0, 0)))
    rel_packed = (rel_packed.reshape(4, REL_ROWS, DIM)
                  .transpose(1, 0, 2).reshape(REL_ROWS, 4 * DIM))

    shape = (NUM_WORKERS, CHUNKS, CHUNK)
    return _SC_KERNEL(h_idx.reshape(shape), r_idx.reshape(shape),
                      t_idx.reshape(shape), ent_packed, rel_packed)


# speedup vs baseline: 1.7659x; 1.0063x over previous
import functools

import jax
import jax.numpy as jnp
from jax import lax
from jax.experimental import pallas as pl
from jax.experimental.pallas import tpu as pltpu
from jax.experimental.pallas import tpu_sc as plsc

NUM_WORKERS = 32
TOTAL = 32768
PER_W = TOTAL // NUM_WORKERS
CHUNKS = 8
CHUNK = PER_W // CHUNKS
DIM = 32

NUM_ENT = 1_000_000
PACK_B = 2048
SUPER = 4 * PACK_B
NSUPER = -(-NUM_ENT // SUPER)
PACK_STEP = 2
NSTEPS = -(-NSUPER // PACK_STEP)
ENT_ROWS = NSTEPS * PACK_STEP * PACK_B
REL_ROWS = 256


def _vsqrt(x):
    b = lax.bitcast_convert_type(x, jnp.int32)
    y = lax.bitcast_convert_type(
        jnp.int32(0x1FBD1DF5) + lax.shift_right_logical(b, 1), jnp.float32)
    for _ in range(3):
        y = 0.5 * (y + x / y)
    return y


def _pack_block(in_ref, out_ref):
    x = in_ref[...]
    eye = (lax.broadcasted_iota(jnp.int32, (DIM, DIM), 0)
           == lax.broadcasted_iota(jnp.int32, (DIM, DIM), 1)
           ).astype(jnp.float32)
    halves = []
    for h in range(PACK_STEP):
        quads = [
            lax.dot_general(
                x[:, h * SUPER + q * PACK_B: h * SUPER + (q + 1) * PACK_B],
                eye, (((0,), (0,)), ((), ())),
                preferred_element_type=jnp.float32)
            for q in range(4)
        ]
        halves.append(jnp.concatenate(quads, axis=1))
    out_ref[...] = jnp.concatenate(halves, axis=0)


def _pack_entities(ent_t):
    return pl.pallas_call(
        _pack_block,
        grid=(NSTEPS,),
        in_specs=[pl.BlockSpec((DIM, PACK_STEP * SUPER), lambda s: (0, s))],
        out_specs=pl.BlockSpec((PACK_STEP * PACK_B, 4 * DIM),
                               lambda s: (s, 0)),
        out_shape=jax.ShapeDtypeStruct((ENT_ROWS, 4 * DIM), jnp.float32),
    )(ent_t)


def _make_sc_kernel():
    mesh = plsc.VectorSubcoreMesh(core_axis_name="c", subcore_axis_name="s")

    @functools.partial(
        pl.kernel,
        mesh=mesh,
        out_type=jax.ShapeDtypeStruct((TOTAL,), jnp.float32),
        compiler_params=pltpu.CompilerParams(
            use_tc_tiling_on_sc=True, needs_layout_passes=False),
        scratch_types=[
            pltpu.VMEM((CHUNKS, CHUNK), jnp.int32),
            pltpu.VMEM((CHUNKS, CHUNK), jnp.int32),
            pltpu.VMEM((CHUNKS, CHUNK), jnp.int32),
            pltpu.VMEM((CHUNKS, CHUNK), jnp.int32),
            pltpu.VMEM((CHUNKS, CHUNK), jnp.int32),
            pltpu.VMEM((CHUNKS, CHUNK), jnp.int32),
            pltpu.VMEM((CHUNKS, CHUNK), jnp.int32),
            pltpu.VMEM((CHUNKS, CHUNK), jnp.int32),
            pltpu.VMEM((CHUNKS, CHUNK), jnp.int32),
            pltpu.VMEM((CHUNK, 4 * DIM), jnp.float32),
            pltpu.VMEM((CHUNK, 4 * DIM), jnp.float32),
            pltpu.VMEM((CHUNK, 4 * DIM), jnp.float32),
            pltpu.VMEM((CHUNK, 4 * DIM), jnp.float32),
            pltpu.VMEM((CHUNK, 4 * DIM), jnp.float32),
            pltpu.VMEM((CHUNK, 4 * DIM), jnp.float32),
            pltpu.VMEM((PER_W,), jnp.float32),
            pltpu.SemaphoreType.DMA,
            pltpu.SemaphoreType.DMA,
        ],
    )
    def kern(h_idx_hbm, r_idx_hbm, t_idx_hbm, entp_hbm, relp_hbm, out_hbm,
             h_idx, r_idx, t_idx, h_row, r_row, t_row, h_cb, r_cb, t_cb,
             h_a, t_a, r_a, h_b, t_b, r_b, out_v, sem_a, sem_b):
        wid = lax.axis_index("s") * 2 + lax.axis_index("c")

        pltpu.sync_copy(h_idx_hbm.at[wid], h_idx)
        pltpu.sync_copy(r_idx_hbm.at[wid], r_idx)
        pltpu.sync_copy(t_idx_hbm.at[wid], t_idx)

        for j in range(CHUNKS):
            for g in range(CHUNK // 16):
                sl = pl.ds(g * 16, 16)
                for src, row, cb in ((h_idx, h_row, h_cb),
                                     (t_idx, t_row, t_cb)):
                    e = src[j, sl]
                    row[j, sl] = ((e >> 13) << 11) + (e & 2047)
                    cb[j, sl] = ((e >> 11) & 3) << 5
                e = r_idx[j, sl]
                r_row[j, sl] = e & 255
                r_cb[j, sl] = ((e >> 8) & 3) << 5

        bufs = ((h_a, t_a, r_a), (h_b, t_b, r_b))
        sems = (sem_a, sem_b)

        def fire(j):
            hb, tb, rb = bufs[j % 2]
            sem = sems[j % 2]
            return (pltpu.async_copy(entp_hbm.at[h_row.at[j]], hb, sem),
                    pltpu.async_copy(entp_hbm.at[t_row.at[j]], tb, sem),
                    pltpu.async_copy(relp_hbm.at[r_row.at[j]], rb, sem))

        lane = lax.iota(jnp.int32, 16)

        def compute(j):
            hb, tb, rb = bufs[j % 2]

            def group(g, _):
                pos = g * 16 + lane
                jv = jnp.full((16,), j, jnp.int32)
                cbh = plsc.load_gather(h_cb, [jv, pos])
                cbt = plsc.load_gather(t_cb, [jv, pos])
                cbr = plsc.load_gather(r_cb, [jv, pos])
                acc = jnp.zeros((16,), jnp.float32)
                for d in range(DIM):
                    h = plsc.load_gather(hb, [pos, cbh + d])
                    t = plsc.load_gather(tb, [pos, cbt + d])
                    r = plsc.load_gather(rb, [pos, cbr + d])
                    dd = (h + r) - t
                    acc = acc + dd * dd
                out_v[pl.ds(j * CHUNK + g * 16, 16)] = _vsqrt(acc)
                return 0

            lax.fori_loop(0, CHUNK // 16, group, 0)

        pending = fire(0)
        for j in range(1, CHUNKS):
            nxt = fire(j)
            for c in pending:
                c.wait()
            compute(j - 1)
            pending = nxt
        for c in pending:
            c.wait()
        compute(CHUNKS - 1)

        pltpu.sync_copy(out_v, out_hbm.at[pl.ds(wid * PER_W, PER_W)])

    return kern


_SC_KERNEL = _make_sc_kernel()


def kernel(pos_h, pos_r, pos_t, neg_h, neg_r, neg_t, entity_emb, relation_emb):
    h_idx = jnp.concatenate([pos_h, neg_h]).astype(jnp.int32)
    r_idx = jnp.concatenate([pos_r, neg_r]).astype(jnp.int32)
    t_idx = jnp.concatenate([pos_t, neg_t]).astype(jnp.int32)

    ent_packed = _pack_entities(entity_emb.T)

    nrel = relation_emb.shape[0]
    rel_packed = jnp.pad(relation_emb, ((0, 4 * REL_ROWS - nrel), (0, 0)))
    rel_packed = (rel_packed.reshape(4, REL_ROWS, DIM)
                  .transpose(1, 0, 2).reshape(REL_ROWS, 4 * DIM))

    shape = (NUM_WORKERS, CHUNKS, CHUNK)
    return _SC_KERNEL(h_idx.reshape(shape), r_idx.reshape(shape),
                      t_idx.reshape(shape), ent_packed, rel_packed)
